# baseline (speedup 1.0000x reference)
import functools

import jax
import jax.numpy as jnp
from jax import lax
from jax.experimental import pallas as pl
from jax.experimental.pallas import tpu as pltpu
from jax.experimental.pallas import tpu_sc as plsc

N = 10000
E = 320000
HID = 128
HALF = 64
NTILES = 16
EPAD = 327680
EPT = EPAD // NTILES
CH = 80
NCHUNK = EPT // CH
NPAD = 10112
RPT = NPAD // NTILES


TCHUNKS = NTILES * NCHUNK


SUPER = 8
NSUPER = NCHUNK // SUPER


def _edge_body(dtab, ebtab, idxpack, out, acc, iring0, iring1, gr0, gr1,
               isem0, isem1, gsem0, gsem1, ssem):
  c = lax.axis_index("c")
  sid = lax.axis_index("s")
  iring = (iring0, iring1)
  grows = (gr0, gr1)
  isem = (isem0, isem1)
  gsem = (gsem0, gsem1)

  zero = jnp.zeros((16,), jnp.float32)

  @plsc.parallel_loop(0, CH)
  def _zrow(rr):
    for i in range(HID // 16):
      gr0[rr, pl.ds(16 * i, 16)] = zero

  @pl.loop(0, RPT // CH)
  def _zcp(j):
    pltpu.sync_copy(gr0.at[pl.ds(0, CH)], acc.at[pl.ds(sid * RPT + j * CH, CH)])

  rem = RPT - (RPT // CH) * CH
  if rem:
    pltpu.sync_copy(gr0.at[pl.ds(0, rem)],
                    acc.at[pl.ds(sid * RPT + (RPT // CH) * CH, rem)])

  plsc.subcore_barrier()

  row0 = c * TCHUNKS + sid * NCHUNK

  def idx_start(kchunk, slot):
    pltpu.async_copy(idxpack.at[pl.ds(row0 + kchunk, SUPER)], iring[slot],
                     isem[slot])

  def idx_wait(kchunk, slot):
    pltpu.make_async_copy(idxpack.at[pl.ds(row0 + kchunk, SUPER)],
                          iring[slot], isem[slot]).wait()

  def gather_start(slot, brow, r):
    pltpu.async_copy(ebtab.at[iring[slot].at[brow, 0]],
                     grows[r].at[pl.ds(0, CH)], gsem[r])
    pltpu.async_copy(dtab.at[iring[slot].at[brow, 1]],
                     grows[r].at[pl.ds(CH, CH)], gsem[r])

  def gather_wait(r):
    pltpu.make_async_copy(ebtab.at[pl.ds(0, 2 * CH)], grows[r],
                          gsem[r]).wait()

  def scatter_start(slot, brow, r):
    pltpu.async_copy(grows[r].at[pl.ds(0, CH)],
                     acc.at[iring[slot].at[brow, 2]], ssem, add=True)

  def scatter_wait(r):
    pltpu.make_async_copy(ebtab.at[pl.ds(0, CH)], grows[r].at[pl.ds(0, CH)],
                          ssem).wait()

  idx_start(0, 0)
  idx_wait(0, 0)
  gather_start(0, 0, 0)

  @pl.loop(0, NCHUNK, step=2 * SUPER)
  def _outer(k0):
    for h in range(2):
      for b in range(SUPER):
        k = k0 + SUPER * h + b
        r = b % 2

        gather_wait(r)

        @pl.when(k >= 1)
        def _():
          scatter_wait(1 - r)

        if b == 1:
          @pl.when(k0 + SUPER * h + SUPER < NCHUNK)
          def _():
            idx_start(k0 + SUPER * h + SUPER, 1 - h)

        if b == SUPER - 1:
          @pl.when(k0 + SUPER * h + SUPER < NCHUNK)
          def _():
            idx_wait(k0 + SUPER * h + SUPER, 1 - h)

        nslot, nbrow = (h, b + 1) if b < SUPER - 1 else (1 - h, 0)

        @pl.when(k + 1 < NCHUNK)
        def _():
          gather_start(nslot, nbrow, 1 - r)

        buf = grows[r]

        @plsc.parallel_loop(0, CH, unroll=2)
        def _edge(e):
          for i in range(HALF // 16):
            d = buf[CH + e, pl.ds(16 * i, 16)]
            eh = buf[e, pl.ds(16 * i, 16)]
            bv = buf[e, pl.ds(HALF + 16 * i, 16)]
            s = 1.0 / (1.0 + jnp.exp(-(d + eh)))
            buf[e, pl.ds(16 * i, 16)] = s * bv
            buf[e, pl.ds(HALF + 16 * i, 16)] = s

        scatter_start(h, b, r)

  scatter_wait(1)

  plsc.subcore_barrier()
  pltpu.sync_copy(acc.at[pl.ds(sid * RPT, RPT)],
                  out.at[pl.ds(c * NPAD + sid * RPT, RPT)])


_edge_call = pl.kernel(
    _edge_body,
    out_type=jax.ShapeDtypeStruct((2 * NPAD, HID), jnp.float32),
    mesh=plsc.VectorSubcoreMesh(core_axis_name="c", subcore_axis_name="s"),
    scratch_types=[
        pltpu.VMEM_SHARED((NPAD, HID), jnp.float32),
        pltpu.VMEM((SUPER, 3, CH), jnp.int32),
        pltpu.VMEM((SUPER, 3, CH), jnp.int32),
        pltpu.VMEM((2 * CH, HID), jnp.float32),
        pltpu.VMEM((2 * CH, HID), jnp.float32),
    ] + [pltpu.SemaphoreType.DMA] * 5,
)


def _mm(x, w, b):
  return jnp.dot(x, w, preferred_element_type=jnp.float32) + b


def _write_tables(h, aw, ab, bw, bb, dw, db, ew, eb,
                  ah_out, dtab_out, ebtab_out):
  ah_out[...] = _mm(h, aw[...], ab[...])
  bh = _mm(h, bw[...], bb[...])
  dh = _mm(h, dw[...], db[...])
  ehm = _mm(h, ew[...], eb[...])
  dtab_out[0:N, :] = dh
  dtab_out[N:2 * N, :] = jnp.concatenate([dh[:, HALF:HID], dh[:, 0:HALF]],
                                         axis=1)
  ebtab_out[0:N, :] = jnp.concatenate([ehm[:, 0:HALF], bh[:, 0:HALF]], axis=1)
  ebtab_out[N:2 * N, :] = jnp.concatenate([ehm[:, HALF:HID], bh[:, HALF:HID]],
                                          axis=1)


def _tc_emb_body(h0, embw, embb, h_out):
  h_out[...] = _mm(h0[...], embw[...], embb[...])


def _tc_tables_body(h_ref, aw, ab, bw, bb, dw, db, ew, eb,
                    ah_out, dtab_out, ebtab_out):
  _write_tables(h_ref[...], aw, ab, bw, bb, dw, db, ew, eb,
                ah_out, dtab_out, ebtab_out)


def _combine_update(nd_ref, ah_ref, hin_ref, g_ref, b_ref):
  nd = nd_ref[...]
  num = jnp.concatenate([nd[0:N, 0:HALF], nd[NPAD:NPAD + N, 0:HALF]], axis=1)
  den = jnp.concatenate([nd[0:N, HALF:HID], nd[NPAD:NPAD + N, HALF:HID]],
                        axis=1)
  h = ah_ref[...] + num / (den + 1e-6)
  m = jnp.mean(h, axis=0, keepdims=True)
  v = jnp.mean((h - m) * (h - m), axis=0, keepdims=True)
  h = (h - m) / jnp.sqrt(v + 1e-5) * g_ref[...] + b_ref[...]
  return hin_ref[...] + jnp.maximum(h, 0.0)


def _tc_update_body(nd_ref, ah_ref, hin_ref, g_ref, b_ref, h_out):
  h_out[...] = _combine_update(nd_ref, ah_ref, hin_ref, g_ref, b_ref)


def _tc_last_body(nd_ref, ah_ref, hin_ref, g_ref, b_ref,
                  w0, b0, w1, b1, w2, b2, y_out):
  h = _combine_update(nd_ref, ah_ref, hin_ref, g_ref, b_ref)
  y = jnp.maximum(_mm(h, w0[...], b0[...]), 0.0)
  y = jnp.maximum(_mm(y, w1[...], b1[...]), 0.0)
  y_out[...] = _mm(y, w2[...], b2[...])


_tab_shapes = (
    jax.ShapeDtypeStruct((N, HID), jnp.float32),
    jax.ShapeDtypeStruct((2 * N, HID), jnp.float32),
    jax.ShapeDtypeStruct((2 * N, HID), jnp.float32),
)

_h_shape = jax.ShapeDtypeStruct((N, HID), jnp.float32)
_tc_emb = pl.pallas_call(_tc_emb_body, out_shape=_h_shape)
_tc_tables = pl.pallas_call(_tc_tables_body, out_shape=_tab_shapes)
_tc_update = pl.pallas_call(_tc_update_body, out_shape=_h_shape)
_tc_last = pl.pallas_call(
    _tc_last_body, out_shape=jax.ShapeDtypeStruct((N, 10), jnp.float32))


def kernel(h, edge_index, emb_w, emb_b, A_w, A_b, B_w, B_b, D_w, D_b,
           E_w, E_b, bnh_g, bnh_b, bne_g, bne_b,
           mlp0_w, mlp0_b, mlp1_w, mlp1_b, mlp2_w, mlp2_b):
  src = edge_index[0]
  dst = edge_index[1]
  npad = EPAD - E
  src_p = jnp.concatenate([src, jnp.zeros((npad,), jnp.int32)])
  dst_p = jnp.concatenate([dst, jnp.zeros((npad,), jnp.int32)])
  dsts = jnp.concatenate([dst, jnp.full((npad,), NPAD - 1, jnp.int32)])
  coff = jnp.array([[0], [N]], jnp.int32)
  sg = (src_p[None, :] + coff).reshape(2, TCHUNKS, CH)
  dg = (dst_p[None, :] + coff).reshape(2, TCHUNKS, CH)
  ds2 = jnp.broadcast_to(dsts[None, :], (2, EPAD)).reshape(2, TCHUNKS, CH)
  idxpack = jnp.stack([sg, dg, ds2], axis=2).reshape(2 * TCHUNKS, 3, CH)

  hh = _tc_emb(h, emb_w, emb_b)
  for l in range(4):
    ah, dtab, ebtab = _tc_tables(hh, A_w[l], A_b[l], B_w[l], B_b[l],
                                 D_w[l], D_b[l], E_w[l], E_b[l])
    nd = _edge_call(dtab, ebtab, idxpack)
    if l < 3:
      hh = _tc_update(nd, ah, hh, bnh_g[l], bnh_b[l])
    else:
      y = _tc_last(nd, ah, hh, bnh_g[l], bnh_b[l],
                   mlp0_w, mlp0_b, mlp1_w, mlp1_b, mlp2_w, mlp2_b)
  return y


# speedup vs baseline: 1.0751x; 1.0751x over previous
import functools

import jax
import jax.numpy as jnp
from jax import lax
from jax.experimental import pallas as pl
from jax.experimental.pallas import tpu as pltpu
from jax.experimental.pallas import tpu_sc as plsc

N = 10000
E = 320000
HID = 128
HALF = 64
NTILES = 16
EPAD = 327680
EPT = EPAD // NTILES
CH = 80
NCHUNK = EPT // CH
NPAD = 10112
RPT = NPAD // NTILES


TCHUNKS = NTILES * NCHUNK


SUPER = 8
NSUPER = NCHUNK // SUPER


def _edge_body(dtab, ebtab, idxpack, out, acc, iring0, iring1, gr0, gr1,
               isem0, isem1, gsem0, gsem1, dsem0, dsem1, ssem):
  c = lax.axis_index("c")
  sid = lax.axis_index("s")
  iring = (iring0, iring1)
  grows = (gr0, gr1)
  isem = (isem0, isem1)
  gsem = (gsem0, gsem1)
  dsem = (dsem0, dsem1)

  zero = jnp.zeros((16,), jnp.float32)

  @plsc.parallel_loop(0, CH)
  def _zrow(rr):
    for i in range(HID // 16):
      gr0[rr, pl.ds(16 * i, 16)] = zero

  @pl.loop(0, RPT // CH)
  def _zcp(j):
    pltpu.sync_copy(gr0.at[pl.ds(0, CH)], acc.at[pl.ds(sid * RPT + j * CH, CH)])

  rem = RPT - (RPT // CH) * CH
  if rem:
    pltpu.sync_copy(gr0.at[pl.ds(0, rem)],
                    acc.at[pl.ds(sid * RPT + (RPT // CH) * CH, rem)])

  plsc.subcore_barrier()

  row0 = c * TCHUNKS + sid * NCHUNK

  def idx_start(kchunk, slot):
    pltpu.async_copy(idxpack.at[pl.ds(row0 + kchunk, SUPER)], iring[slot],
                     isem[slot])

  def idx_wait(kchunk, slot):
    pltpu.make_async_copy(idxpack.at[pl.ds(row0 + kchunk, SUPER)],
                          iring[slot], isem[slot]).wait()

  def gather_start(slot, brow, r):
    pltpu.async_copy(ebtab.at[iring[slot].at[brow, 0]],
                     grows[r].at[pl.ds(0, CH)], gsem[r])
    pltpu.async_copy(dtab.at[iring[slot].at[brow, 1]],
                     grows[r].at[pl.ds(CH, CH)], dsem[r])

  def gather_wait(slot, brow, r):
    pltpu.make_async_copy(ebtab.at[iring[slot].at[brow, 0]],
                          grows[r].at[pl.ds(0, CH)], gsem[r]).wait()
    pltpu.make_async_copy(dtab.at[iring[slot].at[brow, 1]],
                          grows[r].at[pl.ds(CH, CH)], dsem[r]).wait()

  def scatter_start(slot, brow, r):
    pltpu.async_copy(grows[r].at[pl.ds(0, CH)],
                     acc.at[iring[slot].at[brow, 2]], ssem, add=True)

  def scatter_wait(slot, brow, r):
    pltpu.make_async_copy(grows[r].at[pl.ds(0, CH)],
                          acc.at[iring[slot].at[brow, 2]], ssem).wait()

  idx_start(0, 0)
  idx_wait(0, 0)
  gather_start(0, 0, 0)

  @pl.loop(0, NCHUNK, step=2 * SUPER)
  def _outer(k0):
    for h in range(2):
      for b in range(SUPER):
        k = k0 + SUPER * h + b
        r = b % 2

        gather_wait(h, b, r)

        pslot, pbrow = (h, b - 1) if b > 0 else (1 - h, SUPER - 1)

        @pl.when(k >= 1)
        def _():
          scatter_wait(pslot, pbrow, 1 - r)

        if b == 1:
          @pl.when(k0 + SUPER * h + SUPER < NCHUNK)
          def _():
            idx_start(k0 + SUPER * h + SUPER, 1 - h)

        if b == SUPER - 1:
          @pl.when(k0 + SUPER * h + SUPER < NCHUNK)
          def _():
            idx_wait(k0 + SUPER * h + SUPER, 1 - h)

        nslot, nbrow = (h, b + 1) if b < SUPER - 1 else (1 - h, 0)

        @pl.when(k + 1 < NCHUNK)
        def _():
          gather_start(nslot, nbrow, 1 - r)

        buf = grows[r]

        @plsc.parallel_loop(0, CH, unroll=2)
        def _edge(e):
          for i in range(HALF // 16):
            d = buf[CH + e, pl.ds(16 * i, 16)]
            eh = buf[e, pl.ds(16 * i, 16)]
            bv = buf[e, pl.ds(HALF + 16 * i, 16)]
            s = 1.0 / (1.0 + jnp.exp(-(d + eh)))
            buf[e, pl.ds(16 * i, 16)] = s * bv
            buf[e, pl.ds(HALF + 16 * i, 16)] = s

        scatter_start(h, b, r)

  scatter_wait(1, SUPER - 1, 1)

  plsc.subcore_barrier()
  pltpu.sync_copy(acc.at[pl.ds(sid * RPT, RPT)],
                  out.at[pl.ds(c * NPAD + sid * RPT, RPT)])


_edge_call = pl.kernel(
    _edge_body,
    out_type=jax.ShapeDtypeStruct((2 * NPAD, HID), jnp.float32),
    mesh=plsc.VectorSubcoreMesh(core_axis_name="c", subcore_axis_name="s"),
    scratch_types=[
        pltpu.VMEM_SHARED((NPAD, HID), jnp.float32),
        pltpu.VMEM((SUPER, 3, CH), jnp.int32),
        pltpu.VMEM((SUPER, 3, CH), jnp.int32),
        pltpu.VMEM((2 * CH, HID), jnp.float32),
        pltpu.VMEM((2 * CH, HID), jnp.float32),
    ] + [pltpu.SemaphoreType.DMA] * 7,
)


def _mm(x, w, b):
  return jnp.dot(x, w, preferred_element_type=jnp.float32) + b


def _write_tables(h, aw, ab, bw, bb, dw, db, ew, eb,
                  ah_out, dtab_out, ebtab_out):
  ah_out[...] = _mm(h, aw[...], ab[...])
  bh = _mm(h, bw[...], bb[...])
  dh = _mm(h, dw[...], db[...])
  ehm = _mm(h, ew[...], eb[...])
  dtab_out[0:N, :] = dh
  dtab_out[N:2 * N, :] = jnp.concatenate([dh[:, HALF:HID], dh[:, 0:HALF]],
                                         axis=1)
  ebtab_out[0:N, :] = jnp.concatenate([ehm[:, 0:HALF], bh[:, 0:HALF]], axis=1)
  ebtab_out[N:2 * N, :] = jnp.concatenate([ehm[:, HALF:HID], bh[:, HALF:HID]],
                                          axis=1)


def _tc_emb_body(h0, embw, embb, h_out):
  h_out[...] = _mm(h0[...], embw[...], embb[...])


def _tc_tables_body(h_ref, aw, ab, bw, bb, dw, db, ew, eb,
                    ah_out, dtab_out, ebtab_out):
  _write_tables(h_ref[...], aw, ab, bw, bb, dw, db, ew, eb,
                ah_out, dtab_out, ebtab_out)


def _combine_update(nd_ref, ah_ref, hin_ref, g_ref, b_ref):
  nd = nd_ref[...]
  num = jnp.concatenate([nd[0:N, 0:HALF], nd[NPAD:NPAD + N, 0:HALF]], axis=1)
  den = jnp.concatenate([nd[0:N, HALF:HID], nd[NPAD:NPAD + N, HALF:HID]],
                        axis=1)
  h = ah_ref[...] + num / (den + 1e-6)
  m = jnp.mean(h, axis=0, keepdims=True)
  v = jnp.mean((h - m) * (h - m), axis=0, keepdims=True)
  h = (h - m) / jnp.sqrt(v + 1e-5) * g_ref[...] + b_ref[...]
  return hin_ref[...] + jnp.maximum(h, 0.0)


def _tc_update_body(nd_ref, ah_ref, hin_ref, g_ref, b_ref, h_out):
  h_out[...] = _combine_update(nd_ref, ah_ref, hin_ref, g_ref, b_ref)


def _tc_last_body(nd_ref, ah_ref, hin_ref, g_ref, b_ref,
                  w0, b0, w1, b1, w2, b2, y_out):
  h = _combine_update(nd_ref, ah_ref, hin_ref, g_ref, b_ref)
  y = jnp.maximum(_mm(h, w0[...], b0[...]), 0.0)
  y = jnp.maximum(_mm(y, w1[...], b1[...]), 0.0)
  y_out[...] = _mm(y, w2[...], b2[...])


_tab_shapes = (
    jax.ShapeDtypeStruct((N, HID), jnp.float32),
    jax.ShapeDtypeStruct((2 * N, HID), jnp.float32),
    jax.ShapeDtypeStruct((2 * N, HID), jnp.float32),
)

_h_shape = jax.ShapeDtypeStruct((N, HID), jnp.float32)
_tc_emb = pl.pallas_call(_tc_emb_body, out_shape=_h_shape)
_tc_tables = pl.pallas_call(_tc_tables_body, out_shape=_tab_shapes)
_tc_update = pl.pallas_call(_tc_update_body, out_shape=_h_shape)
_tc_last = pl.pallas_call(
    _tc_last_body, out_shape=jax.ShapeDtypeStruct((N, 10), jnp.float32))


def kernel(h, edge_index, emb_w, emb_b, A_w, A_b, B_w, B_b, D_w, D_b,
           E_w, E_b, bnh_g, bnh_b, bne_g, bne_b,
           mlp0_w, mlp0_b, mlp1_w, mlp1_b, mlp2_w, mlp2_b):
  src = edge_index[0]
  dst = edge_index[1]
  npad = EPAD - E
  src_p = jnp.concatenate([src, jnp.zeros((npad,), jnp.int32)])
  dst_p = jnp.concatenate([dst, jnp.zeros((npad,), jnp.int32)])
  dsts = jnp.concatenate([dst, jnp.full((npad,), NPAD - 1, jnp.int32)])
  coff = jnp.array([[0], [N]], jnp.int32)
  sg = (src_p[None, :] + coff).reshape(2, TCHUNKS, CH)
  dg = (dst_p[None, :] + coff).reshape(2, TCHUNKS, CH)
  ds2 = jnp.broadcast_to(dsts[None, :], (2, EPAD)).reshape(2, TCHUNKS, CH)
  idxpack = jnp.stack([sg, dg, ds2], axis=2).reshape(2 * TCHUNKS, 3, CH)

  hh = _tc_emb(h, emb_w, emb_b)
  for l in range(4):
    ah, dtab, ebtab = _tc_tables(hh, A_w[l], A_b[l], B_w[l], B_b[l],
                                 D_w[l], D_b[l], E_w[l], E_b[l])
    nd = _edge_call(dtab, ebtab, idxpack)
    if l < 3:
      hh = _tc_update(nd, ah, hh, bnh_g[l], bnh_b[l])
    else:
      y = _tc_last(nd, ah, hh, bnh_g[l], bnh_b[l],
                   mlp0_w, mlp0_b, mlp1_w, mlp1_b, mlp2_w, mlp2_b)
  return y


# speedup vs baseline: 1.1679x; 1.0863x over previous
import functools

import jax
import jax.numpy as jnp
from jax import lax
from jax.experimental import pallas as pl
from jax.experimental.pallas import tpu as pltpu
from jax.experimental.pallas import tpu_sc as plsc

N = 10000
E = 320000
HID = 128
HALF = 64
NTILES = 16
EPAD = 327680
EPT = EPAD // NTILES
CH = 80
NCHUNK = EPT // CH
NPAD = 10112
RPT = NPAD // NTILES


TCHUNKS = NTILES * NCHUNK


SUPER = 8
NSUPER = NCHUNK // SUPER


def _edge_body(dtab, ebtab, idxpack, out, acc, iring0, iring1, eb0, eb1,
               dh0, dh1, isem0, isem1, gsem0, gsem1, dsem0, dsem1, ssem):
  c = lax.axis_index("c")
  sid = lax.axis_index("s")
  iring = (iring0, iring1)
  ebr = (eb0, eb1)
  dhr = (dh0, dh1)
  isem = (isem0, isem1)
  gsem = (gsem0, gsem1)
  dsem = (dsem0, dsem1)

  zero = jnp.zeros((16,), jnp.float32)

  @plsc.parallel_loop(0, CH)
  def _zrow(rr):
    for i in range(HID // 16):
      eb0[rr, pl.ds(16 * i, 16)] = zero

  @pl.loop(0, RPT // CH)
  def _zcp(j):
    pltpu.sync_copy(eb0, acc.at[pl.ds(sid * RPT + j * CH, CH)])

  rem = RPT - (RPT // CH) * CH
  if rem:
    pltpu.sync_copy(eb0.at[pl.ds(0, rem)],
                    acc.at[pl.ds(sid * RPT + (RPT // CH) * CH, rem)])

  plsc.subcore_barrier()

  row0 = c * TCHUNKS + sid * NCHUNK

  def idx_start(kchunk, slot):
    pltpu.async_copy(idxpack.at[pl.ds(row0 + kchunk, SUPER)], iring[slot],
                     isem[slot])

  def idx_wait(kchunk, slot):
    pltpu.make_async_copy(idxpack.at[pl.ds(row0 + kchunk, SUPER)],
                          iring[slot], isem[slot]).wait()

  def gather_start(slot, brow, r):
    pltpu.async_copy(ebtab.at[iring[slot].at[brow, 0]], ebr[r], gsem[r])
    pltpu.async_copy(dtab.at[iring[slot].at[brow, 1]], dhr[r], dsem[r])

  def gather_wait(slot, brow, r):
    pltpu.make_async_copy(ebtab.at[iring[slot].at[brow, 0]], ebr[r],
                          gsem[r]).wait()
    pltpu.make_async_copy(dtab.at[iring[slot].at[brow, 1]], dhr[r],
                          dsem[r]).wait()

  def scatter_start(slot, brow, r):
    pltpu.async_copy(ebr[r], acc.at[iring[slot].at[brow, 2]], ssem, add=True)

  def scatter_wait(slot, brow, r):
    pltpu.make_async_copy(ebr[r], acc.at[iring[slot].at[brow, 2]],
                          ssem).wait()

  idx_start(0, 0)
  idx_wait(0, 0)
  gather_start(0, 0, 0)

  @pl.loop(0, NCHUNK, step=2 * SUPER)
  def _outer(k0):
    for h in range(2):
      for b in range(SUPER):
        k = k0 + SUPER * h + b
        r = b % 2

        gather_wait(h, b, r)

        pslot, pbrow = (h, b - 1) if b > 0 else (1 - h, SUPER - 1)

        @pl.when(k >= 1)
        def _():
          scatter_wait(pslot, pbrow, 1 - r)

        if b == 1:
          @pl.when(k0 + SUPER * h + SUPER < NCHUNK)
          def _():
            idx_start(k0 + SUPER * h + SUPER, 1 - h)

        if b == SUPER - 1:
          @pl.when(k0 + SUPER * h + SUPER < NCHUNK)
          def _():
            idx_wait(k0 + SUPER * h + SUPER, 1 - h)

        nslot, nbrow = (h, b + 1) if b < SUPER - 1 else (1 - h, 0)

        @pl.when(k + 1 < NCHUNK)
        def _():
          gather_start(nslot, nbrow, 1 - r)

        buf = ebr[r]
        bufd = dhr[r]

        @plsc.parallel_loop(0, CH, unroll=2)
        def _edge(e):
          for i in range(HALF // 16):
            d = bufd[e, pl.ds(16 * i, 16)]
            eh = buf[e, pl.ds(16 * i, 16)]
            bv = buf[e, pl.ds(HALF + 16 * i, 16)]
            s = 1.0 / (1.0 + jnp.exp(-(d + eh)))
            buf[e, pl.ds(16 * i, 16)] = s * bv
            buf[e, pl.ds(HALF + 16 * i, 16)] = s

        scatter_start(h, b, r)

  scatter_wait(1, SUPER - 1, 1)

  plsc.subcore_barrier()
  pltpu.sync_copy(acc.at[pl.ds(sid * RPT, RPT)],
                  out.at[pl.ds(c * NPAD + sid * RPT, RPT)])


_edge_call = pl.kernel(
    _edge_body,
    out_type=jax.ShapeDtypeStruct((2 * NPAD, HID), jnp.float32),
    mesh=plsc.VectorSubcoreMesh(core_axis_name="c", subcore_axis_name="s"),
    scratch_types=[
        pltpu.VMEM_SHARED((NPAD, HID), jnp.float32),
        pltpu.VMEM((SUPER, 3, CH), jnp.int32),
        pltpu.VMEM((SUPER, 3, CH), jnp.int32),
        pltpu.VMEM((CH, HID), jnp.float32),
        pltpu.VMEM((CH, HID), jnp.float32),
        pltpu.VMEM((CH, HALF), jnp.float32),
        pltpu.VMEM((CH, HALF), jnp.float32),
    ] + [pltpu.SemaphoreType.DMA] * 7,
    compiler_params=pltpu.CompilerParams(use_tc_tiling_on_sc=False),
)


def _mm(x, w, b):
  return jnp.dot(x, w, preferred_element_type=jnp.float32) + b


def _write_tables(h, aw, ab, bw, bb, dw, db, ew, eb,
                  ah_out, dtab_out, ebtab_out):
  ah_out[...] = _mm(h, aw[...], ab[...])
  bh = _mm(h, bw[...], bb[...])
  dh = _mm(h, dw[...], db[...])
  ehm = _mm(h, ew[...], eb[...])
  dtab_out[0:N, :] = dh[:, 0:HALF]
  dtab_out[N:2 * N, :] = dh[:, HALF:HID]
  ebtab_out[0:N, :] = jnp.concatenate([ehm[:, 0:HALF], bh[:, 0:HALF]], axis=1)
  ebtab_out[N:2 * N, :] = jnp.concatenate([ehm[:, HALF:HID], bh[:, HALF:HID]],
                                          axis=1)


def _tc_emb_body(h0, embw, embb, h_out):
  h_out[...] = _mm(h0[...], embw[...], embb[...])


def _tc_tables_body(h_ref, aw, ab, bw, bb, dw, db, ew, eb,
                    ah_out, dtab_out, ebtab_out):
  _write_tables(h_ref[...], aw, ab, bw, bb, dw, db, ew, eb,
                ah_out, dtab_out, ebtab_out)


def _combine_update(nd_ref, ah_ref, hin_ref, g_ref, b_ref):
  nd = nd_ref[...]
  num = jnp.concatenate([nd[0:N, 0:HALF], nd[NPAD:NPAD + N, 0:HALF]], axis=1)
  den = jnp.concatenate([nd[0:N, HALF:HID], nd[NPAD:NPAD + N, HALF:HID]],
                        axis=1)
  h = ah_ref[...] + num / (den + 1e-6)
  m = jnp.mean(h, axis=0, keepdims=True)
  v = jnp.mean((h - m) * (h - m), axis=0, keepdims=True)
  h = (h - m) / jnp.sqrt(v + 1e-5) * g_ref[...] + b_ref[...]
  return hin_ref[...] + jnp.maximum(h, 0.0)


def _tc_update_body(nd_ref, ah_ref, hin_ref, g_ref, b_ref, h_out):
  h_out[...] = _combine_update(nd_ref, ah_ref, hin_ref, g_ref, b_ref)


def _tc_last_body(nd_ref, ah_ref, hin_ref, g_ref, b_ref,
                  w0, b0, w1, b1, w2, b2, y_out):
  h = _combine_update(nd_ref, ah_ref, hin_ref, g_ref, b_ref)
  y = jnp.maximum(_mm(h, w0[...], b0[...]), 0.0)
  y = jnp.maximum(_mm(y, w1[...], b1[...]), 0.0)
  y_out[...] = _mm(y, w2[...], b2[...])


_tab_shapes = (
    jax.ShapeDtypeStruct((N, HID), jnp.float32),
    jax.ShapeDtypeStruct((2 * N, HALF), jnp.float32),
    jax.ShapeDtypeStruct((2 * N, HID), jnp.float32),
)

_h_shape = jax.ShapeDtypeStruct((N, HID), jnp.float32)
_tc_emb = pl.pallas_call(_tc_emb_body, out_shape=_h_shape)
_tc_tables = pl.pallas_call(_tc_tables_body, out_shape=_tab_shapes)
_tc_update = pl.pallas_call(_tc_update_body, out_shape=_h_shape)
_tc_last = pl.pallas_call(
    _tc_last_body, out_shape=jax.ShapeDtypeStruct((N, 10), jnp.float32))


def kernel(h, edge_index, emb_w, emb_b, A_w, A_b, B_w, B_b, D_w, D_b,
           E_w, E_b, bnh_g, bnh_b, bne_g, bne_b,
           mlp0_w, mlp0_b, mlp1_w, mlp1_b, mlp2_w, mlp2_b):
  src = edge_index[0]
  dst = edge_index[1]
  npad = EPAD - E
  src_p = jnp.concatenate([src, jnp.zeros((npad,), jnp.int32)])
  dst_p = jnp.concatenate([dst, jnp.zeros((npad,), jnp.int32)])
  dsts = jnp.concatenate([dst, jnp.full((npad,), NPAD - 1, jnp.int32)])
  coff = jnp.array([[0], [N]], jnp.int32)
  sg = (src_p[None, :] + coff).reshape(2, TCHUNKS, CH)
  dg = (dst_p[None, :] + coff).reshape(2, TCHUNKS, CH)
  ds2 = jnp.broadcast_to(dsts[None, :], (2, EPAD)).reshape(2, TCHUNKS, CH)
  idxpack = jnp.stack([sg, dg, ds2], axis=2).reshape(2 * TCHUNKS, 3, CH)

  hh = _tc_emb(h, emb_w, emb_b)
  for l in range(4):
    ah, dtab, ebtab = _tc_tables(hh, A_w[l], A_b[l], B_w[l], B_b[l],
                                 D_w[l], D_b[l], E_w[l], E_b[l])
    nd = _edge_call(dtab, ebtab, idxpack)
    if l < 3:
      hh = _tc_update(nd, ah, hh, bnh_g[l], bnh_b[l])
    else:
      y = _tc_last(nd, ah, hh, bnh_g[l], bnh_b[l],
                   mlp0_w, mlp0_b, mlp1_w, mlp1_b, mlp2_w, mlp2_b)
  return y


# speedup vs baseline: 1.5338x; 1.3133x over previous
import functools

import jax
import jax.numpy as jnp
from jax import lax
from jax.experimental import pallas as pl
from jax.experimental.pallas import tpu as pltpu
from jax.experimental.pallas import tpu_sc as plsc

N = 10000
E = 320000
HID = 128
HALF = 64
NTILES = 16
EPAD = 327680
EPT = EPAD // NTILES
CH = 80
NCHUNK = EPT // CH
NPAD = 10112
RPT = NPAD // NTILES


TCHUNKS = NTILES * NCHUNK


SUPER = 8
NSUPER = NCHUNK // SUPER


def _edge_body(dtab, ebtab, idxpack, out, acc, iring0, iring1, eb0, eb1,
               dh0, dh1, ct0, ct1, isem0, isem1, gsem0, gsem1, dsem0, dsem1,
               ssem):
  c = lax.axis_index("c")
  sid = lax.axis_index("s")
  iring = (iring0, iring1)
  ebr = (eb0, eb1)
  dhr = (dh0, dh1)
  ctr = (ct0, ct1)
  isem = (isem0, isem1)
  gsem = (gsem0, gsem1)
  dsem = (dsem0, dsem1)

  zero = jnp.zeros((16,), jnp.float32)

  @plsc.parallel_loop(0, CH)
  def _zrow(rr):
    for i in range(HID // 16):
      ct0[rr, pl.ds(16 * i, 16)] = zero

  @pl.loop(0, RPT // CH)
  def _zcp(j):
    pltpu.sync_copy(ct0, acc.at[pl.ds(sid * RPT + j * CH, CH)])

  rem = RPT - (RPT // CH) * CH
  if rem:
    pltpu.sync_copy(ct0.at[pl.ds(0, rem)],
                    acc.at[pl.ds(sid * RPT + (RPT // CH) * CH, rem)])

  plsc.subcore_barrier()

  row0 = c * TCHUNKS + sid * NCHUNK

  def idx_start(kchunk, slot):
    pltpu.async_copy(idxpack.at[pl.ds(row0 + kchunk, SUPER)], iring[slot],
                     isem[slot])

  def idx_wait(kchunk, slot):
    pltpu.make_async_copy(idxpack.at[pl.ds(row0 + kchunk, SUPER)],
                          iring[slot], isem[slot]).wait()

  def gather_start(slot, brow, r):
    pltpu.async_copy(ebtab.at[iring[slot].at[brow, 0]], ebr[r], gsem[r])
    pltpu.async_copy(dtab.at[iring[slot].at[brow, 1]], dhr[r], dsem[r])

  def gather_wait(slot, brow, r):
    pltpu.make_async_copy(ebtab.at[iring[slot].at[brow, 0]], ebr[r],
                          gsem[r]).wait()
    pltpu.make_async_copy(dtab.at[iring[slot].at[brow, 1]], dhr[r],
                          dsem[r]).wait()

  def scatter_start(slot, brow, r):
    pltpu.async_copy(ctr[r], acc.at[iring[slot].at[brow, 2]], ssem, add=True)

  def scatter_wait(slot, brow, r):
    pltpu.make_async_copy(ctr[r], acc.at[iring[slot].at[brow, 2]],
                          ssem).wait()

  idx_start(0, 0)
  idx_wait(0, 0)
  gather_start(0, 0, 0)

  @pl.loop(0, NCHUNK, step=2 * SUPER)
  def _outer(k0):
    for h in range(2):
      for b in range(SUPER):
        k = k0 + SUPER * h + b
        r = b % 2

        gather_wait(h, b, r)

        pslot, pbrow = (h, b - 1) if b > 0 else (1 - h, SUPER - 1)

        @pl.when(k >= 1)
        def _():
          scatter_wait(pslot, pbrow, 1 - r)

        if b == 1:
          @pl.when(k0 + SUPER * h + SUPER < NCHUNK)
          def _():
            idx_start(k0 + SUPER * h + SUPER, 1 - h)

        if b == SUPER - 1:
          @pl.when(k0 + SUPER * h + SUPER < NCHUNK)
          def _():
            idx_wait(k0 + SUPER * h + SUPER, 1 - h)

        nslot, nbrow = (h, b + 1) if b < SUPER - 1 else (1 - h, 0)

        @pl.when(k + 1 < NCHUNK)
        def _():
          gather_start(nslot, nbrow, 1 - r)

        buf = ebr[r]
        bufd = dhr[r]
        bufc = ctr[r]

        @plsc.parallel_loop(0, CH, unroll=2)
        def _edge(e):
          for i in range(HALF // 32):
            dpair = plsc.unpack(bufd[e, pl.ds(32 * i, 32)],
                                format=plsc.PackFormat.INTERLEAVED)
            epair = plsc.unpack(buf[e, pl.ds(32 * i, 32)],
                                format=plsc.PackFormat.INTERLEAVED)
            bpair = plsc.unpack(buf[e, pl.ds(HALF + 32 * i, 32)],
                                format=plsc.PackFormat.INTERLEAVED)
            for j in range(2):
              s = 1.0 / (1.0 + jnp.exp(-(dpair[j] + epair[j])))
              bufc[e, pl.ds(32 * i + 16 * j, 16)] = s * bpair[j]
              bufc[e, pl.ds(HALF + 32 * i + 16 * j, 16)] = s

        scatter_start(h, b, r)

  scatter_wait(1, SUPER - 1, 1)

  plsc.subcore_barrier()
  pltpu.sync_copy(acc.at[pl.ds(sid * RPT, RPT)],
                  out.at[pl.ds(c * NPAD + sid * RPT, RPT)])


_edge_call = pl.kernel(
    _edge_body,
    out_type=jax.ShapeDtypeStruct((2 * NPAD, HID), jnp.float32),
    mesh=plsc.VectorSubcoreMesh(core_axis_name="c", subcore_axis_name="s"),
    scratch_types=[
        pltpu.VMEM_SHARED((NPAD, HID), jnp.float32),
        pltpu.VMEM((SUPER, 3, CH), jnp.int32),
        pltpu.VMEM((SUPER, 3, CH), jnp.int32),
        pltpu.VMEM((CH, HID), jnp.bfloat16),
        pltpu.VMEM((CH, HID), jnp.bfloat16),
        pltpu.VMEM((CH, HALF), jnp.bfloat16),
        pltpu.VMEM((CH, HALF), jnp.bfloat16),
        pltpu.VMEM((CH, HID), jnp.float32),
        pltpu.VMEM((CH, HID), jnp.float32),
    ] + [pltpu.SemaphoreType.DMA] * 7,
    compiler_params=pltpu.CompilerParams(use_tc_tiling_on_sc=False,
                                         needs_layout_passes=False),
)


def _mm(x, w, b):
  return jnp.dot(x, w, preferred_element_type=jnp.float32) + b


def _write_tables(h, aw, ab, bw, bb, dw, db, ew, eb,
                  ah_out, dtab_out, ebtab_out):
  ah_out[...] = _mm(h, aw[...], ab[...])
  bh = _mm(h, bw[...], bb[...])
  dh = _mm(h, dw[...], db[...])
  ehm = _mm(h, ew[...], eb[...])
  dtab_out[0:N, :] = dh[:, 0:HALF]
  dtab_out[N:2 * N, :] = dh[:, HALF:HID]
  ebtab_out[0:N, :] = jnp.concatenate([ehm[:, 0:HALF], bh[:, 0:HALF]], axis=1)
  ebtab_out[N:2 * N, :] = jnp.concatenate([ehm[:, HALF:HID], bh[:, HALF:HID]],
                                          axis=1)


def _tc_emb_body(h0, embw, embb, h_out):
  h_out[...] = _mm(h0[...], embw[...], embb[...])


def _tc_tables_body(h_ref, aw, ab, bw, bb, dw, db, ew, eb,
                    ah_out, dtab_out, ebtab_out):
  _write_tables(h_ref[...], aw, ab, bw, bb, dw, db, ew, eb,
                ah_out, dtab_out, ebtab_out)


def _combine_update(nd_ref, ah_ref, hin_ref, g_ref, b_ref):
  nd = nd_ref[...]
  num = jnp.concatenate([nd[0:N, 0:HALF], nd[NPAD:NPAD + N, 0:HALF]], axis=1)
  den = jnp.concatenate([nd[0:N, HALF:HID], nd[NPAD:NPAD + N, HALF:HID]],
                        axis=1)
  h = ah_ref[...] + num / (den + 1e-6)
  m = jnp.mean(h, axis=0, keepdims=True)
  v = jnp.mean((h - m) * (h - m), axis=0, keepdims=True)
  h = (h - m) / jnp.sqrt(v + 1e-5) * g_ref[...] + b_ref[...]
  return hin_ref[...] + jnp.maximum(h, 0.0)


def _tc_update_body(nd_ref, ah_ref, hin_ref, g_ref, b_ref, h_out):
  h_out[...] = _combine_update(nd_ref, ah_ref, hin_ref, g_ref, b_ref)


def _tc_last_body(nd_ref, ah_ref, hin_ref, g_ref, b_ref,
                  w0, b0, w1, b1, w2, b2, y_out):
  h = _combine_update(nd_ref, ah_ref, hin_ref, g_ref, b_ref)
  y = jnp.maximum(_mm(h, w0[...], b0[...]), 0.0)
  y = jnp.maximum(_mm(y, w1[...], b1[...]), 0.0)
  y_out[...] = _mm(y, w2[...], b2[...])


_tab_shapes = (
    jax.ShapeDtypeStruct((N, HID), jnp.float32),
    jax.ShapeDtypeStruct((2 * N, HALF), jnp.float32),
    jax.ShapeDtypeStruct((2 * N, HID), jnp.float32),
)

_h_shape = jax.ShapeDtypeStruct((N, HID), jnp.float32)
_tc_emb = pl.pallas_call(_tc_emb_body, out_shape=_h_shape)
_tc_tables = pl.pallas_call(_tc_tables_body, out_shape=_tab_shapes)
_tc_update = pl.pallas_call(_tc_update_body, out_shape=_h_shape)
_tc_last = pl.pallas_call(
    _tc_last_body, out_shape=jax.ShapeDtypeStruct((N, 10), jnp.float32))


def kernel(h, edge_index, emb_w, emb_b, A_w, A_b, B_w, B_b, D_w, D_b,
           E_w, E_b, bnh_g, bnh_b, bne_g, bne_b,
           mlp0_w, mlp0_b, mlp1_w, mlp1_b, mlp2_w, mlp2_b):
  src = edge_index[0]
  dst = edge_index[1]
  npad = EPAD - E
  src_p = jnp.concatenate([src, jnp.zeros((npad,), jnp.int32)])
  dst_p = jnp.concatenate([dst, jnp.zeros((npad,), jnp.int32)])
  dsts = jnp.concatenate([dst, jnp.full((npad,), NPAD - 1, jnp.int32)])
  coff = jnp.array([[0], [N]], jnp.int32)
  sg = (src_p[None, :] + coff).reshape(2, TCHUNKS, CH)
  dg = (dst_p[None, :] + coff).reshape(2, TCHUNKS, CH)
  ds2 = jnp.broadcast_to(dsts[None, :], (2, EPAD)).reshape(2, TCHUNKS, CH)
  idxpack = jnp.stack([sg, dg, ds2], axis=2).reshape(2 * TCHUNKS, 3, CH)

  def _ilv16(x):
    r, w = x.shape
    xi = x.reshape(r, w // 32, 2, 16).swapaxes(-1, -2).reshape(r, w)
    return xi.astype(jnp.bfloat16)

  hh = _tc_emb(h, emb_w, emb_b)
  for l in range(4):
    ah, dtab, ebtab = _tc_tables(hh, A_w[l], A_b[l], B_w[l], B_b[l],
                                 D_w[l], D_b[l], E_w[l], E_b[l])
    nd = _edge_call(_ilv16(dtab), _ilv16(ebtab), idxpack)
    if l < 3:
      hh = _tc_update(nd, ah, hh, bnh_g[l], bnh_b[l])
    else:
      y = _tc_last(nd, ah, hh, bnh_g[l], bnh_b[l],
                   mlp0_w, mlp0_b, mlp1_w, mlp1_b, mlp2_w, mlp2_b)
  return y


# speedup vs baseline: 1.6606x; 1.0827x over previous
import functools

import jax
import jax.numpy as jnp
from jax import lax
from jax.experimental import pallas as pl
from jax.experimental.pallas import tpu as pltpu
from jax.experimental.pallas import tpu_sc as plsc

N = 10000
E = 320000
HID = 128
HALF = 64
NTILES = 16
EPAD = 327680
EPT = EPAD // NTILES
CH = 128
NCHUNK = EPT // CH
NPAD = 10112
RPT = NPAD // NTILES


TCHUNKS = NTILES * NCHUNK


SUPER = 4
NSUPER = NCHUNK // SUPER


def _edge_body(dtab, ebtab, idxpack, out, acc, iring0, iring1, eb0, eb1,
               dh0, dh1, ct0, isem0, isem1, gsem0, gsem1, dsem0, dsem1,
               ssem):
  c = lax.axis_index("c")
  sid = lax.axis_index("s")
  iring = (iring0, iring1)
  ebr = (eb0, eb1)
  dhr = (dh0, dh1)

  isem = (isem0, isem1)
  gsem = (gsem0, gsem1)
  dsem = (dsem0, dsem1)

  zero = jnp.zeros((16,), jnp.float32)

  @plsc.parallel_loop(0, CH)
  def _zrow(rr):
    for i in range(HID // 16):
      ct0[rr, pl.ds(16 * i, 16)] = zero

  @pl.loop(0, RPT // CH)
  def _zcp(j):
    pltpu.sync_copy(ct0, acc.at[pl.ds(sid * RPT + j * CH, CH)])

  rem = RPT - (RPT // CH) * CH
  if rem:
    pltpu.sync_copy(ct0.at[pl.ds(0, rem)],
                    acc.at[pl.ds(sid * RPT + (RPT // CH) * CH, rem)])

  plsc.subcore_barrier()

  row0 = c * TCHUNKS + sid * NCHUNK

  def idx_start(kchunk, slot):
    pltpu.async_copy(idxpack.at[pl.ds(row0 + kchunk, SUPER)], iring[slot],
                     isem[slot])

  def idx_wait(kchunk, slot):
    pltpu.make_async_copy(idxpack.at[pl.ds(row0 + kchunk, SUPER)],
                          iring[slot], isem[slot]).wait()

  def gather_start(slot, brow, r):
    pltpu.async_copy(ebtab.at[iring[slot].at[brow, 0]], ebr[r], gsem[r])
    pltpu.async_copy(dtab.at[iring[slot].at[brow, 1]], dhr[r], dsem[r])

  def gather_wait(slot, brow, r):
    pltpu.make_async_copy(ebtab.at[iring[slot].at[brow, 0]], ebr[r],
                          gsem[r]).wait()
    pltpu.make_async_copy(dtab.at[iring[slot].at[brow, 1]], dhr[r],
                          dsem[r]).wait()

  def scatter_start(slot, brow):
    pltpu.async_copy(ct0, acc.at[iring[slot].at[brow, 2]], ssem, add=True)

  def scatter_wait(slot, brow):
    pltpu.make_async_copy(ct0, acc.at[iring[slot].at[brow, 2]],
                          ssem).wait()

  idx_start(0, 0)
  idx_wait(0, 0)
  gather_start(0, 0, 0)

  @pl.loop(0, NCHUNK, step=2 * SUPER)
  def _outer(k0):
    for h in range(2):
      for b in range(SUPER):
        k = k0 + SUPER * h + b
        r = b % 2

        gather_wait(h, b, r)

        pslot, pbrow = (h, b - 1) if b > 0 else (1 - h, SUPER - 1)

        @pl.when(k >= 1)
        def _():
          scatter_wait(pslot, pbrow)

        if b == 1:
          @pl.when(k0 + SUPER * h + SUPER < NCHUNK)
          def _():
            idx_start(k0 + SUPER * h + SUPER, 1 - h)

        if b == SUPER - 1:
          @pl.when(k0 + SUPER * h + SUPER < NCHUNK)
          def _():
            idx_wait(k0 + SUPER * h + SUPER, 1 - h)

        nslot, nbrow = (h, b + 1) if b < SUPER - 1 else (1 - h, 0)

        @pl.when(k + 1 < NCHUNK)
        def _():
          gather_start(nslot, nbrow, 1 - r)

        buf = ebr[r]
        bufd = dhr[r]
        bufc = ct0

        @plsc.parallel_loop(0, CH, unroll=2)
        def _edge(e):
          for i in range(HALF // 32):
            dpair = plsc.unpack(bufd[e, pl.ds(32 * i, 32)],
                                format=plsc.PackFormat.INTERLEAVED)
            epair = plsc.unpack(buf[e, pl.ds(32 * i, 32)],
                                format=plsc.PackFormat.INTERLEAVED)
            bpair = plsc.unpack(buf[e, pl.ds(HALF + 32 * i, 32)],
                                format=plsc.PackFormat.INTERLEAVED)
            for j in range(2):
              s = 1.0 / (1.0 + jnp.exp(-(dpair[j] + epair[j])))
              bufc[e, pl.ds(32 * i + 16 * j, 16)] = s * bpair[j]
              bufc[e, pl.ds(HALF + 32 * i + 16 * j, 16)] = s

        scatter_start(h, b)

  scatter_wait(1, SUPER - 1)

  plsc.subcore_barrier()
  pltpu.sync_copy(acc.at[pl.ds(sid * RPT, RPT)],
                  out.at[pl.ds(c * NPAD + sid * RPT, RPT)])


_edge_call = pl.kernel(
    _edge_body,
    out_type=jax.ShapeDtypeStruct((2 * NPAD, HID), jnp.float32),
    mesh=plsc.VectorSubcoreMesh(core_axis_name="c", subcore_axis_name="s"),
    scratch_types=[
        pltpu.VMEM_SHARED((NPAD, HID), jnp.float32),
        pltpu.VMEM((SUPER, 3, CH), jnp.int32),
        pltpu.VMEM((SUPER, 3, CH), jnp.int32),
        pltpu.VMEM((CH, HID), jnp.bfloat16),
        pltpu.VMEM((CH, HID), jnp.bfloat16),
        pltpu.VMEM((CH, HALF), jnp.bfloat16),
        pltpu.VMEM((CH, HALF), jnp.bfloat16),
        pltpu.VMEM((CH, HID), jnp.float32),
    ] + [pltpu.SemaphoreType.DMA] * 7,
    compiler_params=pltpu.CompilerParams(use_tc_tiling_on_sc=False,
                                         needs_layout_passes=False),
)


def _mm(x, w, b):
  return jnp.dot(x, w, preferred_element_type=jnp.float32) + b


def _write_tables(h, aw, ab, bw, bb, dw, db, ew, eb,
                  ah_out, dtab_out, ebtab_out):
  ah_out[...] = _mm(h, aw[...], ab[...])
  bh = _mm(h, bw[...], bb[...])
  dh = _mm(h, dw[...], db[...])
  ehm = _mm(h, ew[...], eb[...])
  dtab_out[0:N, :] = dh[:, 0:HALF]
  dtab_out[N:2 * N, :] = dh[:, HALF:HID]
  ebtab_out[0:N, :] = jnp.concatenate([ehm[:, 0:HALF], bh[:, 0:HALF]], axis=1)
  ebtab_out[N:2 * N, :] = jnp.concatenate([ehm[:, HALF:HID], bh[:, HALF:HID]],
                                          axis=1)


def _tc_emb_body(h0, embw, embb, h_out):
  h_out[...] = _mm(h0[...], embw[...], embb[...])


def _tc_tables_body(h_ref, aw, ab, bw, bb, dw, db, ew, eb,
                    ah_out, dtab_out, ebtab_out):
  _write_tables(h_ref[...], aw, ab, bw, bb, dw, db, ew, eb,
                ah_out, dtab_out, ebtab_out)


def _combine_update(nd_ref, ah_ref, hin_ref, g_ref, b_ref):
  nd = nd_ref[...]
  num = jnp.concatenate([nd[0:N, 0:HALF], nd[NPAD:NPAD + N, 0:HALF]], axis=1)
  den = jnp.concatenate([nd[0:N, HALF:HID], nd[NPAD:NPAD + N, HALF:HID]],
                        axis=1)
  h = ah_ref[...] + num / (den + 1e-6)
  m = jnp.mean(h, axis=0, keepdims=True)
  v = jnp.mean((h - m) * (h - m), axis=0, keepdims=True)
  h = (h - m) / jnp.sqrt(v + 1e-5) * g_ref[...] + b_ref[...]
  return hin_ref[...] + jnp.maximum(h, 0.0)


def _tc_update_body(nd_ref, ah_ref, hin_ref, g_ref, b_ref, h_out):
  h_out[...] = _combine_update(nd_ref, ah_ref, hin_ref, g_ref, b_ref)


def _tc_last_body(nd_ref, ah_ref, hin_ref, g_ref, b_ref,
                  w0, b0, w1, b1, w2, b2, y_out):
  h = _combine_update(nd_ref, ah_ref, hin_ref, g_ref, b_ref)
  y = jnp.maximum(_mm(h, w0[...], b0[...]), 0.0)
  y = jnp.maximum(_mm(y, w1[...], b1[...]), 0.0)
  y_out[...] = _mm(y, w2[...], b2[...])


_tab_shapes = (
    jax.ShapeDtypeStruct((N, HID), jnp.float32),
    jax.ShapeDtypeStruct((2 * N, HALF), jnp.float32),
    jax.ShapeDtypeStruct((2 * N, HID), jnp.float32),
)

_h_shape = jax.ShapeDtypeStruct((N, HID), jnp.float32)
_tc_emb = pl.pallas_call(_tc_emb_body, out_shape=_h_shape)
_tc_tables = pl.pallas_call(_tc_tables_body, out_shape=_tab_shapes)
_tc_update = pl.pallas_call(_tc_update_body, out_shape=_h_shape)
_tc_last = pl.pallas_call(
    _tc_last_body, out_shape=jax.ShapeDtypeStruct((N, 10), jnp.float32))


def kernel(h, edge_index, emb_w, emb_b, A_w, A_b, B_w, B_b, D_w, D_b,
           E_w, E_b, bnh_g, bnh_b, bne_g, bne_b,
           mlp0_w, mlp0_b, mlp1_w, mlp1_b, mlp2_w, mlp2_b):
  src = edge_index[0]
  dst = edge_index[1]
  npad = EPAD - E
  src_p = jnp.concatenate([src, jnp.zeros((npad,), jnp.int32)])
  dst_p = jnp.concatenate([dst, jnp.zeros((npad,), jnp.int32)])
  dsts = jnp.concatenate([dst, jnp.full((npad,), NPAD - 1, jnp.int32)])
  coff = jnp.array([[0], [N]], jnp.int32)
  sg = (src_p[None, :] + coff).reshape(2, TCHUNKS, CH)
  dg = (dst_p[None, :] + coff).reshape(2, TCHUNKS, CH)
  ds2 = jnp.broadcast_to(dsts[None, :], (2, EPAD)).reshape(2, TCHUNKS, CH)
  idxpack = jnp.stack([sg, dg, ds2], axis=2).reshape(2 * TCHUNKS, 3, CH)

  def _ilv16(x):
    r, w = x.shape
    xi = x.reshape(r, w // 32, 2, 16).swapaxes(-1, -2).reshape(r, w)
    return xi.astype(jnp.bfloat16)

  hh = _tc_emb(h, emb_w, emb_b)
  for l in range(4):
    ah, dtab, ebtab = _tc_tables(hh, A_w[l], A_b[l], B_w[l], B_b[l],
                                 D_w[l], D_b[l], E_w[l], E_b[l])
    nd = _edge_call(_ilv16(dtab), _ilv16(ebtab), idxpack)
    if l < 3:
      hh = _tc_update(nd, ah, hh, bnh_g[l], bnh_b[l])
    else:
      y = _tc_last(nd, ah, hh, bnh_g[l], bnh_b[l],
                   mlp0_w, mlp0_b, mlp1_w, mlp1_b, mlp2_w, mlp2_b)
  return y


# speedup vs baseline: 1.7810x; 1.0725x over previous
import functools

import jax
import jax.numpy as jnp
from jax import lax
from jax.experimental import pallas as pl
from jax.experimental.pallas import tpu as pltpu
from jax.experimental.pallas import tpu_sc as plsc

N = 10000
E = 320000
HID = 128
HALF = 64
NTILES = 16
EPAD = 327680
EPT = EPAD // NTILES
CH = 128
NCHUNK = EPT // CH
NPAD = 10112
RPT = NPAD // NTILES


TCHUNKS = NTILES * NCHUNK


SUPER = 4
NSUPER = NCHUNK // SUPER


def _edge_body(dtab, ebtab, idxpack, out, acc, iring0, iring1, eb0, eb1,
               dh0, dh1, ct0, isem0, isem1, gsem0, gsem1, dsem0, dsem1,
               ssem):
  c = lax.axis_index("c")
  sid = lax.axis_index("s")
  iring = (iring0, iring1)
  ebr = (eb0, eb1)
  dhr = (dh0, dh1)

  isem = (isem0, isem1)
  gsem = (gsem0, gsem1)
  dsem = (dsem0, dsem1)

  zero = jnp.zeros((16,), jnp.float32)

  @plsc.parallel_loop(0, CH)
  def _zrow(rr):
    for i in range(HID // 16):
      ct0[rr, pl.ds(16 * i, 16)] = zero

  @pl.loop(0, RPT // CH)
  def _zcp(j):
    pltpu.sync_copy(ct0, acc.at[pl.ds(sid * RPT + j * CH, CH)])

  rem = RPT - (RPT // CH) * CH
  if rem:
    pltpu.sync_copy(ct0.at[pl.ds(0, rem)],
                    acc.at[pl.ds(sid * RPT + (RPT // CH) * CH, rem)])

  plsc.subcore_barrier()

  row0 = c * TCHUNKS + sid * NCHUNK

  def idx_start(kchunk, slot):
    pltpu.async_copy(idxpack.at[pl.ds(row0 + kchunk, SUPER)], iring[slot],
                     isem[slot])

  def idx_wait(kchunk, slot):
    pltpu.make_async_copy(idxpack.at[pl.ds(row0 + kchunk, SUPER)],
                          iring[slot], isem[slot]).wait()

  def gather_start(slot, brow, r):
    pltpu.async_copy(ebtab.at[iring[slot].at[brow, 0]], ebr[r], gsem[r])
    pltpu.async_copy(dtab.at[iring[slot].at[brow, 1]], dhr[r], dsem[r])

  def gather_wait(slot, brow, r):
    pltpu.make_async_copy(ebtab.at[iring[slot].at[brow, 0]], ebr[r],
                          gsem[r]).wait()
    pltpu.make_async_copy(dtab.at[iring[slot].at[brow, 1]], dhr[r],
                          dsem[r]).wait()

  def scatter_start(slot, brow):
    pltpu.async_copy(ct0, acc.at[iring[slot].at[brow, 2]], ssem, add=True)

  def scatter_wait(slot, brow):
    pltpu.make_async_copy(ct0, acc.at[iring[slot].at[brow, 2]],
                          ssem).wait()

  idx_start(0, 0)
  idx_wait(0, 0)
  gather_start(0, 0, 0)

  @pl.loop(0, NCHUNK, step=2 * SUPER)
  def _outer(k0):
    for h in range(2):
      for b in range(SUPER):
        k = k0 + SUPER * h + b
        r = b % 2

        gather_wait(h, b, r)

        pslot, pbrow = (h, b - 1) if b > 0 else (1 - h, SUPER - 1)

        @pl.when(k >= 1)
        def _():
          scatter_wait(pslot, pbrow)

        if b == 1:
          @pl.when(k0 + SUPER * h + SUPER < NCHUNK)
          def _():
            idx_start(k0 + SUPER * h + SUPER, 1 - h)

        if b == SUPER - 1:
          @pl.when(k0 + SUPER * h + SUPER < NCHUNK)
          def _():
            idx_wait(k0 + SUPER * h + SUPER, 1 - h)

        nslot, nbrow = (h, b + 1) if b < SUPER - 1 else (1 - h, 0)

        @pl.when(k + 1 < NCHUNK)
        def _():
          gather_start(nslot, nbrow, 1 - r)

        buf = ebr[r]
        bufd = dhr[r]
        bufc = ct0

        @plsc.parallel_loop(0, CH, unroll=2)
        def _edge(e):
          for i in range(HALF // 32):
            dpair = plsc.unpack(bufd[e, pl.ds(32 * i, 32)],
                                format=plsc.PackFormat.INTERLEAVED)
            epair = plsc.unpack(buf[e, pl.ds(32 * i, 32)],
                                format=plsc.PackFormat.INTERLEAVED)
            bpair = plsc.unpack(buf[e, pl.ds(HALF + 32 * i, 32)],
                                format=plsc.PackFormat.INTERLEAVED)
            for j in range(2):
              s = 1.0 / (1.0 + jnp.exp(-(dpair[j] + epair[j])))
              bufc[e, pl.ds(32 * i + 16 * j, 16)] = s * bpair[j]
              bufc[e, pl.ds(HALF + 32 * i + 16 * j, 16)] = s

        scatter_start(h, b)

  scatter_wait(1, SUPER - 1)

  plsc.subcore_barrier()
  pltpu.sync_copy(acc.at[pl.ds(sid * RPT, RPT)],
                  out.at[pl.ds(c * NPAD + sid * RPT, RPT)])


_edge_call = pl.kernel(
    _edge_body,
    out_type=jax.ShapeDtypeStruct((2 * NPAD, HID), jnp.float32),
    mesh=plsc.VectorSubcoreMesh(core_axis_name="c", subcore_axis_name="s"),
    scratch_types=[
        pltpu.VMEM_SHARED((NPAD, HID), jnp.float32),
        pltpu.VMEM((SUPER, 3, CH), jnp.int32),
        pltpu.VMEM((SUPER, 3, CH), jnp.int32),
        pltpu.VMEM((CH, HID), jnp.bfloat16),
        pltpu.VMEM((CH, HID), jnp.bfloat16),
        pltpu.VMEM((CH, HALF), jnp.bfloat16),
        pltpu.VMEM((CH, HALF), jnp.bfloat16),
        pltpu.VMEM((CH, HID), jnp.float32),
    ] + [pltpu.SemaphoreType.DMA] * 7,
    compiler_params=pltpu.CompilerParams(use_tc_tiling_on_sc=False,
                                         needs_layout_passes=False),
)


_ILV_COLS = jnp.array(
    [32 * (j // 32) + 16 * (j % 2) + (j % 32) // 2 for j in range(HID)],
    jnp.int32)


def _mm(x, w, b):
  return jnp.dot(x, w, preferred_element_type=jnp.float32) + b


def _write_tables(h, aw, ab, bw, bb, dw, db, ew, eb,
                  ah_out, dtab_out, ebtab_out):
  ah_out[...] = _mm(h, aw[...], ab[...])
  bh = _mm(h, bw[...], bb[...])
  dh = _mm(h, dw[...], db[...])
  ehm = _mm(h, ew[...], eb[...])
  bf = jnp.bfloat16
  dtab_out[0:N, :] = dh[:, 0:HALF].astype(bf)
  dtab_out[N:2 * N, :] = dh[:, HALF:HID].astype(bf)
  ebtab_out[0:N, :] = jnp.concatenate([ehm[:, 0:HALF], bh[:, 0:HALF]],
                                      axis=1).astype(bf)
  ebtab_out[N:2 * N, :] = jnp.concatenate([ehm[:, HALF:HID], bh[:, HALF:HID]],
                                          axis=1).astype(bf)


def _tc_emb_body(h0, embw, embb, h_out):
  h_out[...] = _mm(h0[...], embw[...], embb[...])


def _tc_tables_body(h_ref, aw, ab, bw, bb, dw, db, ew, eb,
                    ah_out, dtab_out, ebtab_out):
  _write_tables(h_ref[...], aw, ab, bw, bb, dw, db, ew, eb,
                ah_out, dtab_out, ebtab_out)


def _combine_update(nd_ref, ah_ref, hin_ref, g_ref, b_ref):
  nd = nd_ref[...]
  num = jnp.concatenate([nd[0:N, 0:HALF], nd[NPAD:NPAD + N, 0:HALF]], axis=1)
  den = jnp.concatenate([nd[0:N, HALF:HID], nd[NPAD:NPAD + N, HALF:HID]],
                        axis=1)
  h = ah_ref[...] + num / (den + 1e-6)
  m = jnp.mean(h, axis=0, keepdims=True)
  v = jnp.mean((h - m) * (h - m), axis=0, keepdims=True)
  h = (h - m) / jnp.sqrt(v + 1e-5) * g_ref[...] + b_ref[...]
  return hin_ref[...] + jnp.maximum(h, 0.0)


def _tc_update_body(nd_ref, ah_ref, hin_ref, g_ref, b_ref, h_out):
  h_out[...] = _combine_update(nd_ref, ah_ref, hin_ref, g_ref, b_ref)


def _tc_last_body(nd_ref, ah_ref, hin_ref, g_ref, b_ref,
                  w0, b0, w1, b1, w2, b2, y_out):
  h = _combine_update(nd_ref, ah_ref, hin_ref, g_ref, b_ref)
  y = jnp.maximum(_mm(h, w0[...], b0[...]), 0.0)
  y = jnp.maximum(_mm(y, w1[...], b1[...]), 0.0)
  y_out[...] = _mm(y, w2[...], b2[...])


_tab_shapes = (
    jax.ShapeDtypeStruct((N, HID), jnp.float32),
    jax.ShapeDtypeStruct((2 * N, HALF), jnp.bfloat16),
    jax.ShapeDtypeStruct((2 * N, HID), jnp.bfloat16),
)

_h_shape = jax.ShapeDtypeStruct((N, HID), jnp.float32)
_tc_emb = pl.pallas_call(_tc_emb_body, out_shape=_h_shape)
_tc_tables = pl.pallas_call(_tc_tables_body, out_shape=_tab_shapes)
_tc_update = pl.pallas_call(_tc_update_body, out_shape=_h_shape)
_tc_last = pl.pallas_call(
    _tc_last_body, out_shape=jax.ShapeDtypeStruct((N, 10), jnp.float32))


def kernel(h, edge_index, emb_w, emb_b, A_w, A_b, B_w, B_b, D_w, D_b,
           E_w, E_b, bnh_g, bnh_b, bne_g, bne_b,
           mlp0_w, mlp0_b, mlp1_w, mlp1_b, mlp2_w, mlp2_b):
  src = edge_index[0]
  dst = edge_index[1]
  npad = EPAD - E
  src_p = jnp.concatenate([src, jnp.zeros((npad,), jnp.int32)])
  dst_p = jnp.concatenate([dst, jnp.zeros((npad,), jnp.int32)])
  dsts = jnp.concatenate([dst, jnp.full((npad,), NPAD - 1, jnp.int32)])
  coff = jnp.array([[0], [N]], jnp.int32)
  sg = (src_p[None, :] + coff).reshape(2, TCHUNKS, CH)
  dg = (dst_p[None, :] + coff).reshape(2, TCHUNKS, CH)
  ds2 = jnp.broadcast_to(dsts[None, :], (2, EPAD)).reshape(2, TCHUNKS, CH)
  idxpack = jnp.stack([sg, dg, ds2], axis=2).reshape(2 * TCHUNKS, 3, CH)

  cm = _ILV_COLS
  B_w, B_b = B_w[:, :, cm], B_b[:, cm]
  D_w, D_b = D_w[:, :, cm], D_b[:, cm]
  E_w, E_b = E_w[:, :, cm], E_b[:, cm]

  hh = _tc_emb(h, emb_w, emb_b)
  for l in range(4):
    ah, dtab, ebtab = _tc_tables(hh, A_w[l], A_b[l], B_w[l], B_b[l],
                                 D_w[l], D_b[l], E_w[l], E_b[l])
    nd = _edge_call(dtab, ebtab, idxpack)
    if l < 3:
      hh = _tc_update(nd, ah, hh, bnh_g[l], bnh_b[l])
    else:
      y = _tc_last(nd, ah, hh, bnh_g[l], bnh_b[l],
                   mlp0_w, mlp0_b, mlp1_w, mlp1_b, mlp2_w, mlp2_b)
  return y
